# baseline (speedup 1.0000x reference)
import functools

import jax
import jax.numpy as jnp
from jax import lax
from jax.experimental import pallas as pl
from jax.experimental.pallas import tpu as pltpu
from jax.experimental.pallas import tpu_sc as plsc

_NUM_CLASSES = 1000000
_FEAT = 64
_BATCH = 16384
_LAMBDA = 0.003

_NC = 2
_NS = 16
_NW = _NC * _NS
_ROWS = _BATCH // _NW
_GCHUNK = 128
_NG = _ROWS // _GCHUNK


def _sc_partials(centers, labels3d, feats3d):
    mesh = plsc.VectorSubcoreMesh(core_axis_name="c", subcore_axis_name="s",
                                  num_cores=_NC, num_subcores=_NS)

    @functools.partial(
        pl.kernel,
        out_type=jax.ShapeDtypeStruct((_NW, 16), jnp.float32),
        mesh=mesh,
        scratch_types=[
            pltpu.VMEM((_NG, _GCHUNK), jnp.int32),
            pltpu.VMEM((_ROWS, _FEAT), jnp.float32),
            pltpu.VMEM((_ROWS, _FEAT), jnp.float32),
            pltpu.VMEM((16,), jnp.float32),
            pltpu.SemaphoreType.DMA,
        ],
        compiler_params=pltpu.CompilerParams(use_tc_tiling_on_sc=False),
    )
    def k(centers_hbm, labels_hbm, feats_hbm, out_hbm, idx_v, cbuf, fbuf, accv, sem):
        wid = lax.axis_index("s") * _NC + lax.axis_index("c")

        pltpu.sync_copy(labels_hbm.at[wid], idx_v)
        copies = [
            pltpu.async_copy(centers_hbm.at[idx_v.at[j]],
                             cbuf.at[pl.ds(j * _GCHUNK, _GCHUNK)], sem)
            for j in range(_NG)
        ]
        pltpu.sync_copy(feats_hbm.at[wid], fbuf)
        for c in copies:
            c.wait()

        zero = jnp.zeros((16,), jnp.float32)

        def body(r, accs):
            out = []
            for c in range(_FEAT // 16):
                d = fbuf[r, pl.ds(c * 16, 16)] - cbuf[r, pl.ds(c * 16, 16)]
                out.append(accs[c] + d * d)
            return tuple(out)

        accs = lax.fori_loop(0, _ROWS, body, (zero,) * (_FEAT // 16))
        accv[...] = (accs[0] + accs[1]) + (accs[2] + accs[3])
        pltpu.sync_copy(accv, out_hbm.at[wid])

    return k(centers, labels3d, feats3d)


def _reduce_body(p_ref, o_ref):
    s = jnp.sum(p_ref[...]) * (_LAMBDA / (2.0 * _BATCH))
    o_ref[...] = s[None, None]


def kernel(features, labels, centers):
    labels3d = labels.reshape(_NW, _NG, _GCHUNK)
    feats3d = features.reshape(_NW, _ROWS, _FEAT)
    partials = _sc_partials(centers, labels3d, feats3d)
    out = pl.pallas_call(
        _reduce_body,
        out_shape=jax.ShapeDtypeStruct((1, 1), jnp.float32),
    )(partials)
    return out[0, 0]


# speedup vs baseline: 1.6883x; 1.6883x over previous
import functools

import jax
import jax.numpy as jnp
from jax import lax
from jax.experimental import pallas as pl
from jax.experimental.pallas import tpu as pltpu
from jax.experimental.pallas import tpu_sc as plsc

_NUM_CLASSES = 1000000
_FEAT = 64
_BATCH = 16384
_LAMBDA = 0.003

_NC = 2
_NS = 16
_NW = _NC * _NS
_ROWS = _BATCH // _NW
_SEG = 256


def _sc_partials(centers, labels1d, features):
    mesh = plsc.VectorSubcoreMesh(core_axis_name="c", subcore_axis_name="s",
                                  num_cores=_NC, num_subcores=_NS)

    @functools.partial(
        pl.kernel,
        out_type=jax.ShapeDtypeStruct((_NW, 128), jnp.float32),
        mesh=mesh,
        scratch_types=[
            pltpu.VMEM((_ROWS,), jnp.int32),
            pltpu.VMEM((_SEG, _FEAT), jnp.float32),
            pltpu.VMEM((_SEG, _FEAT), jnp.float32),
            pltpu.VMEM((128,), jnp.float32),
            pltpu.SemaphoreType.DMA,
            pltpu.SemaphoreType.DMA,
        ],
    )
    def k(c_hbm, labels_hbm, f_hbm, out_hbm, lbuf, rowbuf, fbuf, accv, gsem, fsem):
        wid = lax.axis_index("s") * _NC + lax.axis_index("c")
        base = wid * _ROWS

        pltpu.sync_copy(labels_hbm.at[pl.ds(base, _ROWS)], lbuf)

        acc = jnp.zeros((16,), jnp.float32)
        for seg in range(_ROWS // _SEG):
            fcp = pltpu.async_copy(
                f_hbm.at[pl.ds(base + seg * _SEG, _SEG), :], fbuf, fsem)

            def fire_body(g, _, seg=seg):
                lv = lbuf[pl.ds(seg * _SEG + g * 16, 16)]
                for j in range(16):
                    pltpu.async_copy(c_hbm.at[lv[j]], rowbuf.at[g * 16 + j],
                                     gsem)
                return 0

            lax.fori_loop(0, _SEG // 16, fire_body, 0)
            fcp.wait()

            def comp_body(r, acc):
                pltpu.make_async_copy(c_hbm.at[0], rowbuf.at[0], gsem).wait()
                for c in range(_FEAT // 16):
                    fv = fbuf[r, pl.ds(c * 16, 16)]
                    cv = rowbuf[r, pl.ds(c * 16, 16)]
                    d = fv - cv
                    acc = acc + d * d
                return acc

            acc = lax.fori_loop(0, _SEG, comp_body, acc)

        zero16 = jnp.zeros((16,), jnp.float32)
        for i in range(8):
            accv[pl.ds(i * 16, 16)] = acc if i == 0 else zero16
        pltpu.sync_copy(accv, out_hbm.at[wid])

    return k(centers, labels1d, features)


def _reduce_body(p_ref, o_ref):
    s = jnp.sum(p_ref[...]) * (_LAMBDA / (2.0 * _BATCH))
    o_ref[...] = s[None, None]


def kernel(features, labels, centers):
    labels1d = labels.reshape(_BATCH)
    partials = _sc_partials(centers, labels1d, features)
    out = pl.pallas_call(
        _reduce_body,
        out_shape=jax.ShapeDtypeStruct((1, 1), jnp.float32),
    )(partials)
    return out[0, 0]
